# baseline (device time: 301172 ns/iter reference)
import jax
import jax.numpy as jnp
from jax import lax
from jax.experimental import pallas as pl
from jax.experimental.pallas import tpu as pltpu

N_DEV = 4


def kernel(x, w_mat):
    x = x.astype(jnp.bfloat16)
    w_mat = w_mat.astype(jnp.bfloat16)

    m_per, k = x.shape
    _, n_per = w_mat.shape

    def body(x_ref, w_ref, out_ref, comm_ref, send_sems, recv_sems,
             amax_ref, amax_send_sems, amax_recv_sems):
        my = lax.axis_index("i")
        left = (my - 1) % N_DEV
        right = (my + 1) % N_DEV

        barrier_sem = pltpu.get_barrier_semaphore()
        for nbr in (left, right):
            pl.semaphore_signal(
                barrier_sem, inc=1,
                device_id=(nbr,), device_id_type=pl.DeviceIdType.MESH,
            )
        pl.semaphore_wait(barrier_sem, 2)

        def make_hop(h):
            src = x_ref if h == 0 else comm_ref.at[h - 1]
            return pltpu.make_async_remote_copy(
                src_ref=src,
                dst_ref=comm_ref.at[h],
                send_sem=send_sems.at[h],
                recv_sem=recv_sems.at[h],
                device_id=(right,),
                device_id_type=pl.DeviceIdType.MESH,
            )

        hops = [make_hop(0)]
        hops[0].start()

        out_ref[pl.ds(my * m_per, m_per), :] = jnp.dot(
            x_ref[...], w_ref[...], preferred_element_type=jnp.float32)

        for h in range(N_DEV - 1):
            hops[h].wait()
            if h + 1 < N_DEV - 1:
                nxt = make_hop(h + 1)
                nxt.start()
                hops.append(nxt)
            origin = (my - h - 1) % N_DEV
            out_ref[pl.ds(origin * m_per, m_per), :] = jnp.dot(
                comm_ref[h], w_ref[...], preferred_element_type=jnp.float32)

        local_amax = jnp.max(jnp.abs(out_ref[...]))
        amax_ref[pl.ds(my, 1), :] = jnp.full((1, 128), local_amax,
                                             jnp.float32)

        sends = []
        for d in range(1, N_DEV):
            peer = (my + d) % N_DEV
            s = pltpu.make_async_remote_copy(
                src_ref=amax_ref.at[pl.ds(my, 1)],
                dst_ref=amax_ref.at[pl.ds(my, 1)],
                send_sem=amax_send_sems.at[d - 1],
                recv_sem=amax_recv_sems.at[d - 1],
                device_id=(peer,),
                device_id_type=pl.DeviceIdType.MESH,
            )
            s.start()
            sends.append(s)

        for d in range(1, N_DEV):
            origin = (my - d) % N_DEV
            r = pltpu.make_async_remote_copy(
                src_ref=amax_ref.at[pl.ds(origin, 1)],
                dst_ref=amax_ref.at[pl.ds(origin, 1)],
                send_sem=amax_send_sems.at[d - 1],
                recv_sem=amax_recv_sems.at[d - 1],
                device_id=(origin,),
                device_id_type=pl.DeviceIdType.MESH,
            )
            r.wait_recv()
        for s in sends:
            s.wait_send()

        global_amax = jnp.max(amax_ref[...])
        scale = global_amax / 127.0
        inv_scale = 127.0 / global_amax
        y = out_ref[...]
        q = jnp.clip(jnp.round(y * inv_scale), -127.0, 127.0)
        out_ref[...] = q * scale

    return pl.pallas_call(
        body,
        out_shape=jax.ShapeDtypeStruct((N_DEV * m_per, n_per), jnp.float32),
        in_specs=[
            pl.BlockSpec(memory_space=pltpu.VMEM),
            pl.BlockSpec(memory_space=pltpu.VMEM),
        ],
        out_specs=pl.BlockSpec(memory_space=pltpu.VMEM),
        scratch_shapes=[
            pltpu.VMEM((N_DEV - 1, m_per, k), jnp.bfloat16),
            pltpu.SemaphoreType.DMA((N_DEV - 1,)),
            pltpu.SemaphoreType.DMA((N_DEV - 1,)),
            pltpu.VMEM((N_DEV, 128), jnp.float32),
            pltpu.SemaphoreType.DMA((N_DEV - 1,)),
            pltpu.SemaphoreType.DMA((N_DEV - 1,)),
        ],
        compiler_params=pltpu.CompilerParams(collective_id=0),
    )(x, w_mat)


# device time: 165801 ns/iter; 1.8165x vs baseline; 1.8165x over previous
import functools

import jax
import jax.numpy as jnp
from jax import lax
from jax.experimental import pallas as pl
from jax.experimental.pallas import tpu as pltpu

N_DEV = 4


def kernel(x, w_mat):
    x = x.astype(jnp.bfloat16)
    w_mat = w_mat.astype(jnp.bfloat16)

    m_per, k = x.shape
    _, n_per = w_mat.shape
    m_half = m_per // 2

    def body(x_ref, w_ref, out_ref, commR_ref, commL_ref,
             sendR_sems, recvR_sems, sendL_sems, recvL_sems,
             amax_ref, amax_send_sems, amax_recv_sems):
        my = lax.axis_index("i")
        left = (my - 1) % N_DEV
        right = (my + 1) % N_DEV

        barrier_sem = pltpu.get_barrier_semaphore()
        for nbr in (left, right):
            pl.semaphore_signal(
                barrier_sem, inc=1,
                device_id=(nbr,), device_id_type=pl.DeviceIdType.MESH,
            )
        pl.semaphore_wait(barrier_sem, 2)

        def make_hopR(h):
            src = x_ref.at[pl.ds(0, m_half)] if h == 0 else commR_ref.at[h - 1]
            return pltpu.make_async_remote_copy(
                src_ref=src,
                dst_ref=commR_ref.at[h],
                send_sem=sendR_sems.at[h],
                recv_sem=recvR_sems.at[h],
                device_id=(right,),
                device_id_type=pl.DeviceIdType.MESH,
            )

        def make_hopL(h):
            src = (x_ref.at[pl.ds(m_half, m_half)] if h == 0
                   else commL_ref.at[h - 1])
            return pltpu.make_async_remote_copy(
                src_ref=src,
                dst_ref=commL_ref.at[h],
                send_sem=sendL_sems.at[h],
                recv_sem=recvL_sems.at[h],
                device_id=(left,),
                device_id_type=pl.DeviceIdType.MESH,
            )

        hopsR = [make_hopR(0)]
        hopsL = [make_hopL(0)]
        hopsR[0].start()
        hopsL[0].start()

        amax_parts = []

        def gemm(src, row0):
            yblk = jnp.dot(src, w_ref[...],
                           preferred_element_type=jnp.float32)
            out_ref[pl.ds(row0, yblk.shape[0]), :] = yblk
            amax_parts.append(jnp.max(jnp.abs(yblk)))

        gemm(x_ref[...], my * m_per)

        for h in range(N_DEV - 1):
            hopsR[h].wait_recv()
            hopsL[h].wait_recv()
            if h + 1 < N_DEV - 1:
                hopsR.append(make_hopR(h + 1))
                hopsL.append(make_hopL(h + 1))
                hopsR[h + 1].start()
                hopsL[h + 1].start()
            originR = (my - h - 1) % N_DEV
            originL = (my + h + 1) % N_DEV
            gemm(commR_ref[h], originR * m_per)
            gemm(commL_ref[h], originL * m_per + m_half)

        for hop in hopsR + hopsL:
            hop.wait_send()

        local_amax = functools.reduce(jnp.maximum, amax_parts)
        amax_ref[pl.ds(my, 1), :] = jnp.full((1, 128), local_amax,
                                             jnp.float32)

        sends = []
        for d in range(1, N_DEV):
            peer = (my + d) % N_DEV
            s = pltpu.make_async_remote_copy(
                src_ref=amax_ref.at[pl.ds(my, 1)],
                dst_ref=amax_ref.at[pl.ds(my, 1)],
                send_sem=amax_send_sems.at[d - 1],
                recv_sem=amax_recv_sems.at[d - 1],
                device_id=(peer,),
                device_id_type=pl.DeviceIdType.MESH,
            )
            s.start()
            sends.append(s)

        for d in range(1, N_DEV):
            origin = (my - d) % N_DEV
            r = pltpu.make_async_remote_copy(
                src_ref=amax_ref.at[pl.ds(origin, 1)],
                dst_ref=amax_ref.at[pl.ds(origin, 1)],
                send_sem=amax_send_sems.at[d - 1],
                recv_sem=amax_recv_sems.at[d - 1],
                device_id=(origin,),
                device_id_type=pl.DeviceIdType.MESH,
            )
            r.wait_recv()
        for s in sends:
            s.wait_send()

        global_amax = jnp.max(amax_ref[...])
        scale = global_amax / 127.0
        inv_scale = 127.0 / global_amax
        y = out_ref[...]
        q = jnp.clip(jnp.round(y * inv_scale), -127.0, 127.0)
        out_ref[...] = q * scale

    return pl.pallas_call(
        body,
        out_shape=jax.ShapeDtypeStruct((N_DEV * m_per, n_per), jnp.float32),
        in_specs=[
            pl.BlockSpec(memory_space=pltpu.VMEM),
            pl.BlockSpec(memory_space=pltpu.VMEM),
        ],
        out_specs=pl.BlockSpec(memory_space=pltpu.VMEM),
        scratch_shapes=[
            pltpu.VMEM((N_DEV - 1, m_half, k), jnp.bfloat16),
            pltpu.VMEM((N_DEV - 1, m_half, k), jnp.bfloat16),
            pltpu.SemaphoreType.DMA((N_DEV - 1,)),
            pltpu.SemaphoreType.DMA((N_DEV - 1,)),
            pltpu.SemaphoreType.DMA((N_DEV - 1,)),
            pltpu.SemaphoreType.DMA((N_DEV - 1,)),
            pltpu.VMEM((N_DEV, 128), jnp.float32),
            pltpu.SemaphoreType.DMA((N_DEV - 1,)),
            pltpu.SemaphoreType.DMA((N_DEV - 1,)),
        ],
        compiler_params=pltpu.CompilerParams(collective_id=0),
    )(x, w_mat)
